# baseline (speedup 1.0000x reference)
import functools

import jax
import jax.numpy as jnp
from jax import lax
from jax.experimental import pallas as pl
from jax.experimental.pallas import tpu as pltpu
from jax.experimental.pallas import tpu_sc as plsc

_BATCH = 16384
_FEAT = 1024
_GROUPS = 64
_NC = 2
_NS = 16
_NW = _NC * _NS
_RPW = _BATCH // _NW
_C = 32
_NCHUNK = _RPW // _C

_mesh = plsc.VectorSubcoreMesh(core_axis_name="c", subcore_axis_name="s")


@functools.partial(
    pl.kernel,
    out_type=jax.ShapeDtypeStruct((_BATCH, _FEAT), jnp.float32),
    mesh=_mesh,
    scratch_types=[
        pltpu.VMEM((_C,), jnp.int32),
        pltpu.VMEM((_C, _FEAT), jnp.float32),
        pltpu.VMEM((_C, _FEAT), jnp.float32),
        pltpu.VMEM((_C, _FEAT), jnp.float32),
        pltpu.SemaphoreType.DMA,
    ],
)
def _sc_norm(x_hbm, gidx_hbm, mean_hbm, std_hbm, out_hbm,
             idx_v, x_v, m_v, s_v, sem):
    wid = lax.axis_index("s") * _NC + lax.axis_index("c")
    base = wid * _RPW

    def chunk(ci, carry):
        b = base + ci * _C
        pltpu.sync_copy(gidx_hbm.at[pl.ds(b, _C)], idx_v)
        pltpu.async_copy(x_hbm.at[pl.ds(b, _C)], x_v, sem).wait()
        pltpu.async_copy(mean_hbm.at[idx_v], m_v, sem).wait()
        pltpu.async_copy(std_hbm.at[idx_v], s_v, sem).wait()

        def row(i, rcarry):
            for j in range(_FEAT // 16):
                sl = pl.ds(j * 16, 16)
                x_v[i, sl] = (x_v[i, sl] - m_v[i, sl]) / (s_v[i, sl] + 1e-5)
            return rcarry

        lax.fori_loop(0, _C, row, 0)
        pltpu.sync_copy(x_v, out_hbm.at[pl.ds(b, _C)])
        return carry

    lax.fori_loop(0, _NCHUNK, chunk, 0)


def kernel(x, group_idx, mean, std):
    return _sc_norm(x, group_idx.astype(jnp.int32), mean, std)


# speedup vs baseline: 1.1660x; 1.1660x over previous
import functools

import jax
import jax.numpy as jnp
from jax import lax
from jax.experimental import pallas as pl
from jax.experimental.pallas import tpu as pltpu
from jax.experimental.pallas import tpu_sc as plsc

_BATCH = 16384
_FEAT = 1024
_GROUPS = 64
_NC = 2
_NS = 16
_NW = _NC * _NS
_RPW = _BATCH // _NW
_C = 8
_NCHUNK = _RPW // _C
_RIN = 4
_ROUT = 2

_mesh = plsc.VectorSubcoreMesh(core_axis_name="c", subcore_axis_name="s")


def _recip_body(std_ref, out_ref):
    out_ref[...] = 1.0 / (std_ref[...] + 1e-5)


def _recip_table(std):
    return pl.pallas_call(
        _recip_body,
        out_shape=jax.ShapeDtypeStruct((_GROUPS, _FEAT), jnp.float32),
    )(std)


@functools.partial(
    pl.kernel,
    out_type=jax.ShapeDtypeStruct((_BATCH, _FEAT), jnp.float32),
    mesh=_mesh,
    scratch_types=[
        [pltpu.VMEM((_C, _FEAT), jnp.float32) for _ in range(_RIN)],
        [pltpu.VMEM((_C, _FEAT), jnp.float32) for _ in range(_RIN)],
        [pltpu.VMEM((_C, _FEAT), jnp.float32) for _ in range(_RIN)],
        [pltpu.VMEM((_C, _FEAT), jnp.float32) for _ in range(_ROUT)],
        pltpu.VMEM((_RPW,), jnp.int32),
        [pltpu.SemaphoreType.DMA for _ in range(_RIN)],
        [pltpu.SemaphoreType.DMA for _ in range(_ROUT)],
    ],
)
def _sc_norm(x_hbm, gidx_hbm, mean_hbm, rtab_hbm, out_hbm,
             x_v, m_v, r_v, y_v, idx_all,
             insem, outsem):
    sid = lax.axis_index("s")
    wid = sid * _NC + lax.axis_index("c")
    base = wid * _RPW

    pltpu.sync_copy(gidx_hbm.at[pl.ds(base, _RPW)], idx_all)

    def start_in(c, r):
        @pl.when(c < _NCHUNK)
        def _():
            idx_sl = idx_all.at[pl.ds(c * _C, _C)]
            pltpu.async_copy(x_hbm.at[pl.ds(base + c * _C, _C)], x_v[r],
                             insem[r])
            pltpu.async_copy(mean_hbm.at[idx_sl], m_v[r], insem[r])
            pltpu.async_copy(rtab_hbm.at[idx_sl], r_v[r], insem[r])

    def drain_in(r):
        for dst in (x_v[r], m_v[r], r_v[r]):
            pltpu.make_async_copy(x_hbm.at[pl.ds(0, _C)], dst,
                                  insem[r]).wait()

    def wait_out(q):
        pltpu.make_async_copy(x_hbm.at[pl.ds(0, _C)], y_v[q],
                              outsem[q]).wait()

    for r in range(_RIN):
        start_in(r, r)

    def outer(k, carry):
        for r in range(_RIN):
            c = k * _RIN + r
            q = r % _ROUT
            drain_in(r)

            @pl.when(c >= _ROUT)
            def _():
                wait_out(q)

            def row(i, rc):
                for j in range(_FEAT // 16):
                    sl = pl.ds(j * 16, 16)
                    y_v[q][i, sl] = (x_v[r][i, sl] - m_v[r][i, sl]) * r_v[r][i, sl]
                return rc

            lax.fori_loop(0, _C, row, 0, unroll=2)
            pltpu.async_copy(y_v[q], out_hbm.at[pl.ds(base + c * _C, _C)],
                             outsem[q])
            start_in(c + _RIN, r)
        return carry

    lax.fori_loop(0, _NCHUNK // _RIN, outer, 0)
    for q in range(_ROUT):
        wait_out(q)


def kernel(x, group_idx, mean, std):
    rtab = _recip_table(std)
    return _sc_norm(x, group_idx.astype(jnp.int32), mean, rtab)


# speedup vs baseline: 1.2480x; 1.0703x over previous
import functools

import jax
import jax.numpy as jnp
from jax import lax
from jax.experimental import pallas as pl
from jax.experimental.pallas import tpu as pltpu
from jax.experimental.pallas import tpu_sc as plsc

_BATCH = 16384
_FEAT = 1024
_GROUPS = 64
_NC = 2
_NS = 16
_NW = _NC * _NS
_RPW = _BATCH // _NW
_C = 8
_NCHUNK = _RPW // _C
_RIN = 4
_ROUT = 2

_mesh = plsc.VectorSubcoreMesh(core_axis_name="c", subcore_axis_name="s")


def _recip_body(std_ref, out_ref):
    out_ref[...] = 1.0 / (std_ref[...] + 1e-5)


def _recip_table(std):
    return pl.pallas_call(
        _recip_body,
        out_shape=jax.ShapeDtypeStruct((_GROUPS, _FEAT), jnp.float32),
    )(std)


@functools.partial(
    pl.kernel,
    out_type=jax.ShapeDtypeStruct((_BATCH, _FEAT), jnp.float32),
    mesh=_mesh,
    scratch_types=[
        [pltpu.VMEM((_C, _FEAT), jnp.float32) for _ in range(_RIN)],
        [pltpu.VMEM((_C, _FEAT), jnp.float32) for _ in range(_RIN)],
        [pltpu.VMEM((_C, _FEAT), jnp.float32) for _ in range(_RIN)],
        [pltpu.VMEM((_C, _FEAT), jnp.float32) for _ in range(_ROUT)],
        pltpu.VMEM((_RPW,), jnp.int32),
        [pltpu.SemaphoreType.DMA for _ in range(_RIN)],
        [pltpu.SemaphoreType.DMA for _ in range(_ROUT)],
    ],
)
def _sc_norm(x_hbm, gidx_hbm, mean_hbm, rtab_hbm, out_hbm,
             x_v, m_v, r_v, y_v, idx_all,
             insem, outsem):
    sid = lax.axis_index("s")
    wid = sid * _NC + lax.axis_index("c")
    base = wid * _RPW

    pltpu.sync_copy(gidx_hbm.at[pl.ds(base, _RPW)], idx_all)

    def start_in(c, r):
        @pl.when(c < _NCHUNK)
        def _():
            idx_sl = idx_all.at[pl.ds(c * _C, _C)]
            pltpu.async_copy(x_hbm.at[pl.ds(base + c * _C, _C)], x_v[r],
                             insem[r])
            pltpu.async_copy(mean_hbm.at[idx_sl], m_v[r], insem[r])
            pltpu.async_copy(rtab_hbm.at[idx_sl], r_v[r], insem[r])

    def drain_in(r):
        for dst in (x_v[r], m_v[r], r_v[r]):
            pltpu.make_async_copy(x_hbm.at[pl.ds(0, _C)], dst,
                                  insem[r]).wait()

    def wait_out(q):
        pltpu.make_async_copy(x_hbm.at[pl.ds(0, _C)], y_v[q],
                              outsem[q]).wait()

    for r in range(_RIN):
        start_in(r, r)

    def outer(k, carry):
        for r in range(_RIN):
            c = k * _RIN + r
            q = r % _ROUT
            drain_in(r)

            @pl.when(c >= _ROUT)
            def _():
                wait_out(q)

            @plsc.parallel_loop(0, _C, step=1, unroll=2)
            def row(i):
                for j in range(_FEAT // 16):
                    sl = pl.ds(j * 16, 16)
                    y_v[q][i, sl] = (x_v[r][i, sl] - m_v[r][i, sl]) * r_v[r][i, sl]
            pltpu.async_copy(y_v[q], out_hbm.at[pl.ds(base + c * _C, _C)],
                             outsem[q])
            start_in(c + _RIN, r)
        return carry

    lax.fori_loop(0, _NCHUNK // _RIN, outer, 0)
    for q in range(_ROUT):
        wait_out(q)


def kernel(x, group_idx, mean, std):
    rtab = _recip_table(std)
    return _sc_norm(x, group_idx.astype(jnp.int32), mean, rtab)
